# baseline (speedup 1.0000x reference)
import jax
import jax.numpy as jnp
from jax.experimental import pallas as pl
from jax.experimental.pallas import tpu as pltpu


def _gate_body(x_ref, w_ref, b_ref, o_ref, pooled_ref):
    i = pl.program_id(0)
    xs = x_ref[0]
    s = jnp.sum(xs, axis=1) * (1.0 / xs.shape[1])
    pooled_ref[i, :] = s

    @pl.when(i == pl.num_programs(0) - 1)
    def _():
        pooled = pooled_ref[...]
        logits = jax.lax.dot_general(
            pooled, w_ref[...], (((1,), (1,)), ((), ())),
            preferred_element_type=jnp.float32,
        )
        o_ref[...] = jax.nn.sigmoid(logits + b_ref[...])


def kernel(x, Wc, b):
    B, C, H, W = x.shape
    hw = H * W
    x3 = x.reshape(B, C, hw)
    b2 = b.reshape(1, C)
    out = pl.pallas_call(
        _gate_body,
        grid=(B,),
        in_specs=[
            pl.BlockSpec((1, C, hw), lambda i: (i, 0, 0)),
            pl.BlockSpec((C, C), lambda i: (0, 0)),
            pl.BlockSpec((1, C), lambda i: (0, 0)),
        ],
        out_specs=pl.BlockSpec((B, C), lambda i: (0, 0)),
        out_shape=jax.ShapeDtypeStruct((B, C), jnp.float32),
        scratch_shapes=[pltpu.VMEM((B, C), jnp.float32)],
    )(x3, Wc, b2)
    return out.reshape(B, 1, C, 1, 1)


# speedup vs baseline: 1.1612x; 1.1612x over previous
import jax
import jax.numpy as jnp
from jax.experimental import pallas as pl
from jax.experimental.pallas import tpu as pltpu

_NBUF = 8


def _gate_body(x_hbm, w_ref, b_ref, o_ref, bufs, logits, sems):
    B = x_hbm.shape[0]

    def _copy(b, k):
        return pltpu.make_async_copy(
            x_hbm.at[pl.ds(b, 1)], bufs.at[pl.ds(k, 1)], sems.at[k]
        )

    for k in range(_NBUF):
        _copy(k, k).start()

    def phase(p, carry):
        for k in range(_NBUF):
            b = p * _NBUF + k
            _copy(b, k).wait()
            xs = bufs[k]
            col = jnp.sum(xs, axis=1, keepdims=True)
            row = jax.lax.dot_general(
                col, w_ref[...], (((0,), (1,)), ((), ())),
                preferred_element_type=jnp.float32,
            )
            logits[pl.ds(b, 1), :] = row

            nb = b + _NBUF

            @pl.when(nb < B)
            def _():
                _copy(nb, k).start()
        return carry

    jax.lax.fori_loop(0, B // _NBUF, phase, 0)

    scale = 1.0 / x_hbm.shape[2]
    o_ref[...] = jax.nn.sigmoid(logits[...] * scale + b_ref[...])


def kernel(x, Wc, b):
    B, C, H, W = x.shape
    hw = H * W
    x3 = x.reshape(B, C, hw)
    b2 = b.reshape(1, C)
    out = pl.pallas_call(
        _gate_body,
        in_specs=[
            pl.BlockSpec(memory_space=pl.ANY),
            pl.BlockSpec(memory_space=pltpu.VMEM),
            pl.BlockSpec(memory_space=pltpu.VMEM),
        ],
        out_specs=pl.BlockSpec(memory_space=pltpu.VMEM),
        out_shape=jax.ShapeDtypeStruct((B, C), jnp.float32),
        scratch_shapes=[
            pltpu.VMEM((_NBUF, C, hw), jnp.float32),
            pltpu.VMEM((B, C), jnp.float32),
            pltpu.SemaphoreType.DMA((_NBUF,)),
        ],
    )(x3, Wc, b2)
    return out.reshape(B, 1, C, 1, 1)


# speedup vs baseline: 1.1712x; 1.0086x over previous
import jax
import jax.numpy as jnp
from jax.experimental import pallas as pl
from jax.experimental.pallas import tpu as pltpu

_NOPS = 8
_NBATCH = 4


def _gate_body(*refs):
    x_refs = refs[:_NOPS]
    w_ref, b_ref, o_ref, logits = refs[_NOPS:]
    i = pl.program_id(0)
    for bb in range(_NBATCH):
        col = jnp.concatenate(
            [jnp.sum(xq[bb], axis=1, keepdims=True) for xq in x_refs], axis=0
        )
        row = jax.lax.dot_general(
            col, w_ref[...], (((0,), (1,)), ((), ())),
            preferred_element_type=jnp.float32,
        )
        logits[i * _NBATCH + bb, :] = row[0]

    @pl.when(i == pl.num_programs(0) - 1)
    def _():
        scale = 1.0 / x_refs[0].shape[2]
        o_ref[...] = jax.nn.sigmoid(logits[...] * scale + b_ref[...])


def kernel(x, Wc, b):
    B, C, H, W = x.shape
    hw = H * W
    x3 = x.reshape(B, C, hw)
    b2 = b.reshape(1, C)
    csl = C // _NOPS

    def _xspec(q):
        return pl.BlockSpec((_NBATCH, csl, hw), lambda i, q=q: (i, q, 0))

    out = pl.pallas_call(
        _gate_body,
        grid=(B // _NBATCH,),
        in_specs=[_xspec(q) for q in range(_NOPS)] + [
            pl.BlockSpec((C, C), lambda i: (0, 0)),
            pl.BlockSpec((1, C), lambda i: (0, 0)),
        ],
        out_specs=pl.BlockSpec((B, C), lambda i: (0, 0)),
        out_shape=jax.ShapeDtypeStruct((B, C), jnp.float32),
        scratch_shapes=[pltpu.VMEM((B, C), jnp.float32)],
    )(*([x3] * _NOPS), Wc, b2)
    return out.reshape(B, 1, C, 1, 1)


# speedup vs baseline: 3.0180x; 2.5769x over previous
import jax
import jax.numpy as jnp
from jax.experimental import pallas as pl
from jax.experimental.pallas import tpu as pltpu


def _gate_body(x_ref, w_ref, b_ref, o_ref, pooled_ref):
    i = pl.program_id(0)
    xs = x_ref[0]
    pooled_ref[pl.ds(i, 1), :] = jnp.sum(xs, axis=0, keepdims=True)

    @pl.when(i == pl.num_programs(0) - 1)
    def _():
        pooled = pooled_ref[...]
        logits = jax.lax.dot_general(
            pooled, w_ref[...], (((1,), (1,)), ((), ())),
            preferred_element_type=jnp.float32,
        )
        scale = 1.0 / xs.shape[0]
        o_ref[...] = jax.nn.sigmoid(logits * scale + b_ref[...])


def kernel(x, Wc, b):
    B, C, H, W = x.shape
    hw = H * W
    xt = jnp.transpose(x, (0, 2, 3, 1)).reshape(B, hw, C)
    b2 = b.reshape(1, C)
    out = pl.pallas_call(
        _gate_body,
        grid=(B,),
        in_specs=[
            pl.BlockSpec((1, hw, C), lambda i: (i, 0, 0)),
            pl.BlockSpec((C, C), lambda i: (0, 0)),
            pl.BlockSpec((1, C), lambda i: (0, 0)),
        ],
        out_specs=pl.BlockSpec((B, C), lambda i: (0, 0)),
        out_shape=jax.ShapeDtypeStruct((B, C), jnp.float32),
        scratch_shapes=[pltpu.VMEM((B, C), jnp.float32)],
    )(xt, Wc, b2)
    return out.reshape(B, 1, C, 1, 1)


# speedup vs baseline: 4.2789x; 1.4178x over previous
import jax
import jax.numpy as jnp
from jax.experimental import pallas as pl
from jax.experimental.pallas import tpu as pltpu

_NOPS = 8
_NBATCH = 2


def _gate_body(*refs):
    x_refs = refs[:_NOPS]
    w_ref, b_ref, o_ref, pooled_ref = refs[_NOPS:]
    i = pl.program_id(0)
    for bb in range(_NBATCH):
        parts = [jnp.sum(xq[bb], axis=0, keepdims=True) for xq in x_refs]
        row = parts[0]
        for p in parts[1:]:
            row = row + p
        pooled_ref[pl.ds(i * _NBATCH + bb, 1), :] = row

    @pl.when(i == pl.num_programs(0) - 1)
    def _():
        pooled = pooled_ref[...]
        logits = jax.lax.dot_general(
            pooled, w_ref[...], (((1,), (1,)), ((), ())),
            preferred_element_type=jnp.float32,
        )
        scale = 1.0 / (x_refs[0].shape[1] * _NOPS)
        o_ref[...] = jax.nn.sigmoid(logits * scale + b_ref[...])


def kernel(x, Wc, b):
    B, C, H, W = x.shape
    hw = H * W
    hsl = hw // _NOPS
    xt = jnp.transpose(x, (0, 2, 3, 1)).reshape(B, hw, C)
    b2 = b.reshape(1, C)

    def _xspec(q):
        return pl.BlockSpec((_NBATCH, hsl, C), lambda i, q=q: (i, q, 0))

    out = pl.pallas_call(
        _gate_body,
        grid=(B // _NBATCH,),
        in_specs=[_xspec(q) for q in range(_NOPS)] + [
            pl.BlockSpec((C, C), lambda i: (0, 0)),
            pl.BlockSpec((1, C), lambda i: (0, 0)),
        ],
        out_specs=pl.BlockSpec((B, C), lambda i: (0, 0)),
        out_shape=jax.ShapeDtypeStruct((B, C), jnp.float32),
        scratch_shapes=[pltpu.VMEM((B, C), jnp.float32)],
    )(*([xt] * _NOPS), Wc, b2)
    return out.reshape(B, 1, C, 1, 1)
